# Initial kernel scaffold; baseline (speedup 1.0000x reference)
import jax
import jax.numpy as jnp
from jax.experimental import pallas as pl


def kernel(inputs, table):
    raise NotImplementedError("write your pallas kernel here")


# speedup vs baseline: 4.1328x; 4.1328x over previous
import functools

import jax
import jax.numpy as jnp
from jax import lax
from jax.experimental import pallas as pl
from jax.experimental.pallas import tpu as pltpu
from jax.experimental.pallas import tpu_sc as plsc

DEPTH = 64
MAX_LENGTH = 200
BATCH = 4096
SEQ = 200

NUM_CORES = 2
NUM_SUBCORES = 16
NUM_WORKERS = NUM_CORES * NUM_SUBCORES

SEQ_PER_WORKER = BATCH // NUM_WORKERS
ROWS_PER_CHUNK = SEQ
HALF = ROWS_PER_CHUNK // 2
NBUF = 2


def _positional_encodings(depth: int, max_length: int) -> jnp.ndarray:
    positions = jnp.arange(max_length, dtype=jnp.float32)[:, None]
    idx = jnp.arange(depth)[None, :]
    power = (2 * (idx // 2)).astype(jnp.float32) / jnp.float32(depth)
    angles = 1.0 / jnp.power(10000.0, power)
    radians = positions * angles
    sin = jnp.sin(radians[:, 0::2])
    cos = jnp.cos(radians[:, 1::2])
    return jnp.concatenate([sin, cos], axis=-1)


def _body(idx_hbm, table_hbm, enc_hbm, out_hbm,
          idx_v, enc_v, in0, in1, ob0, ob1, gs0, gs1, os0, os1):
    wid = lax.axis_index("s") * NUM_CORES + lax.axis_index("c")

    pltpu.sync_copy(idx_hbm.at[pl.ds(wid * (2 * SEQ_PER_WORKER), 2 * SEQ_PER_WORKER)],
                    idx_v)
    pltpu.sync_copy(enc_hbm, enc_v)

    in_bufs = (in0, in1)
    out_bufs = (ob0, ob1)
    gsems = (gs0, gs1)
    osems = (os0, os1)

    row_base = wid * (SEQ_PER_WORKER * SEQ)

    def gather_descs(c, buf, sem):
        d0 = pltpu.make_async_copy(table_hbm.at[idx_v.at[2 * c]],
                                   buf.at[pl.ds(0, HALF)], sem)
        d1 = pltpu.make_async_copy(table_hbm.at[idx_v.at[2 * c + 1]],
                                   buf.at[pl.ds(HALF, HALF)], sem)
        return d0, d1

    def issue_gather(c, buf, sem):
        d0, d1 = gather_descs(c, buf, sem)
        d0.start()
        d1.start()

    def wait_gather(c, buf, sem):
        d0, d1 = gather_descs(c, buf, sem)
        d0.wait()
        d1.wait()

    def out_desc(c, buf, sem):
        return pltpu.make_async_copy(
            buf, out_hbm.at[pl.ds(row_base + c * ROWS_PER_CHUNK, ROWS_PER_CHUNK)],
            sem)

    def compute(inb, outb):
        scale = jnp.float32(8.0)

        def row_body(r, carry):
            for d in range(DEPTH // 16):
                sl = pl.ds(d * 16, 16)
                outb[r, sl] = inb[r, sl] * scale + enc_v[r, sl]
            return carry

        lax.fori_loop(0, ROWS_PER_CHUNK, row_body, 0)

    for b in range(NBUF):
        issue_gather(b, in_bufs[b], gsems[b])

    def outer(i, carry):
        for b in range(NBUF):
            c = i * NBUF + b
            wait_gather(c, in_bufs[b], gsems[b])

            @pl.when(c >= NBUF)
            def _():
                out_desc(0, out_bufs[b], osems[b]).wait()

            compute(in_bufs[b], out_bufs[b])

            @pl.when(c + NBUF < SEQ_PER_WORKER)
            def _():
                issue_gather(c + NBUF, in_bufs[b], gsems[b])

            out_desc(c, out_bufs[b], osems[b]).start()
        return carry

    lax.fori_loop(0, SEQ_PER_WORKER // NBUF, outer, 0)

    for b in range(NBUF):
        out_desc(0, out_bufs[b], osems[b]).wait()


@functools.partial(jax.jit, static_argnames=())
def _embed(idx2d, table, enc):
    run = pl.kernel(
        _body,
        out_type=jax.ShapeDtypeStruct((BATCH * SEQ, DEPTH), jnp.float32),
        mesh=plsc.VectorSubcoreMesh(core_axis_name="c", subcore_axis_name="s"),
        scratch_types=[
            pltpu.VMEM((2 * SEQ_PER_WORKER, HALF), jnp.int32),
            pltpu.VMEM((MAX_LENGTH, DEPTH), jnp.float32),
            pltpu.VMEM((ROWS_PER_CHUNK, DEPTH), jnp.float32),
            pltpu.VMEM((ROWS_PER_CHUNK, DEPTH), jnp.float32),
            pltpu.VMEM((ROWS_PER_CHUNK, DEPTH), jnp.float32),
            pltpu.VMEM((ROWS_PER_CHUNK, DEPTH), jnp.float32),
            pltpu.SemaphoreType.DMA,
            pltpu.SemaphoreType.DMA,
            pltpu.SemaphoreType.DMA,
            pltpu.SemaphoreType.DMA,
        ],
        compiler_params=pltpu.CompilerParams(use_tc_tiling_on_sc=False),
    )
    return run(idx2d, table, enc)


def kernel(inputs, table):
    idx2d = inputs.astype(jnp.int32).reshape(BATCH * SEQ // HALF, HALF)
    enc = _positional_encodings(DEPTH, MAX_LENGTH)
    out = _embed(idx2d, table, enc)
    return out.reshape(BATCH, SEQ, DEPTH)


# speedup vs baseline: 4.1399x; 1.0017x over previous
import functools

import jax
import jax.numpy as jnp
from jax import lax
from jax.experimental import pallas as pl
from jax.experimental.pallas import tpu as pltpu
from jax.experimental.pallas import tpu_sc as plsc

DEPTH = 64
MAX_LENGTH = 200
BATCH = 4096
SEQ = 200

NUM_CORES = 2
NUM_SUBCORES = 16
NUM_WORKERS = NUM_CORES * NUM_SUBCORES

SEQ_PER_WORKER = BATCH // NUM_WORKERS
ROWS_PER_CHUNK = SEQ
HALF = ROWS_PER_CHUNK // 2
NBUF = 2


def _positional_encodings(depth: int, max_length: int) -> jnp.ndarray:
    positions = jnp.arange(max_length, dtype=jnp.float32)[:, None]
    idx = jnp.arange(depth)[None, :]
    power = (2 * (idx // 2)).astype(jnp.float32) / jnp.float32(depth)
    angles = 1.0 / jnp.power(10000.0, power)
    radians = positions * angles
    sin = jnp.sin(radians[:, 0::2])
    cos = jnp.cos(radians[:, 1::2])
    return jnp.concatenate([sin, cos], axis=-1)


def _body(idx_hbm, table_hbm, enc_hbm, out_hbm,
          idx_v, enc_v, in0, in1, ob0, ob1, gs0, gs1, os0, os1):
    wid = lax.axis_index("s") * NUM_CORES + lax.axis_index("c")

    pltpu.sync_copy(idx_hbm.at[pl.ds(wid * (2 * SEQ_PER_WORKER), 2 * SEQ_PER_WORKER)],
                    idx_v)
    pltpu.sync_copy(enc_hbm, enc_v)

    in_bufs = (in0, in1)
    out_bufs = (ob0, ob1)
    gsems = (gs0, gs1)
    osems = (os0, os1)

    row_base = wid * (SEQ_PER_WORKER * SEQ)

    def gather_descs(c, buf, sem):
        d0 = pltpu.make_async_copy(table_hbm.at[idx_v.at[2 * c]],
                                   buf.at[pl.ds(0, HALF)], sem)
        d1 = pltpu.make_async_copy(table_hbm.at[idx_v.at[2 * c + 1]],
                                   buf.at[pl.ds(HALF, HALF)], sem)
        return d0, d1

    def issue_gather(c, buf, sem):
        d0, d1 = gather_descs(c, buf, sem)
        d0.start()
        d1.start()

    def wait_gather(c, buf, sem):
        d0, d1 = gather_descs(c, buf, sem)
        d0.wait()
        d1.wait()

    def out_desc(c, buf, sem):
        return pltpu.make_async_copy(
            buf, out_hbm.at[pl.ds(row_base + c * ROWS_PER_CHUNK, ROWS_PER_CHUNK)],
            sem)

    def compute(inb, outb):
        scale = jnp.float32(8.0)

        @plsc.parallel_loop(0, ROWS_PER_CHUNK, 1, unroll=4)
        def _(r):
            for d in range(DEPTH // 16):
                sl = pl.ds(d * 16, 16)
                outb[r, sl] = inb[r, sl] * scale + enc_v[r, sl]

    for b in range(NBUF):
        issue_gather(b, in_bufs[b], gsems[b])

    def outer(i, carry):
        for b in range(NBUF):
            c = i * NBUF + b
            wait_gather(c, in_bufs[b], gsems[b])

            @pl.when(c >= NBUF)
            def _():
                out_desc(0, out_bufs[b], osems[b]).wait()

            compute(in_bufs[b], out_bufs[b])

            @pl.when(c + NBUF < SEQ_PER_WORKER)
            def _():
                issue_gather(c + NBUF, in_bufs[b], gsems[b])

            out_desc(c, out_bufs[b], osems[b]).start()
        return carry

    lax.fori_loop(0, SEQ_PER_WORKER // NBUF, outer, 0)

    for b in range(NBUF):
        out_desc(0, out_bufs[b], osems[b]).wait()


@functools.partial(jax.jit, static_argnames=())
def _embed(idx2d, table, enc):
    run = pl.kernel(
        _body,
        out_type=jax.ShapeDtypeStruct((BATCH * SEQ, DEPTH), jnp.float32),
        mesh=plsc.VectorSubcoreMesh(core_axis_name="c", subcore_axis_name="s"),
        scratch_types=[
            pltpu.VMEM((2 * SEQ_PER_WORKER, HALF), jnp.int32),
            pltpu.VMEM((MAX_LENGTH, DEPTH), jnp.float32),
            pltpu.VMEM((ROWS_PER_CHUNK, DEPTH), jnp.float32),
            pltpu.VMEM((ROWS_PER_CHUNK, DEPTH), jnp.float32),
            pltpu.VMEM((ROWS_PER_CHUNK, DEPTH), jnp.float32),
            pltpu.VMEM((ROWS_PER_CHUNK, DEPTH), jnp.float32),
            pltpu.SemaphoreType.DMA,
            pltpu.SemaphoreType.DMA,
            pltpu.SemaphoreType.DMA,
            pltpu.SemaphoreType.DMA,
        ],
        compiler_params=pltpu.CompilerParams(use_tc_tiling_on_sc=False),
    )
    return run(idx2d, table, enc)


def kernel(inputs, table):
    idx2d = inputs.astype(jnp.int32).reshape(BATCH * SEQ // HALF, HALF)
    enc = _positional_encodings(DEPTH, MAX_LENGTH)
    out = _embed(idx2d, table, enc)
    return out.reshape(BATCH, SEQ, DEPTH)
